# Initial kernel scaffold; baseline (speedup 1.0000x reference)
import jax
import jax.numpy as jnp
from jax.experimental import pallas as pl


def kernel(xyz1, xyz2, points1, points2, dn_w0, dn_b0, dn_w1, dn_b1, dn_w2, dn_b2, wn_w0, wn_b0, wn_w1, wn_b1, wn_w2, wn_b2, lin_w, lin_b, m1_w, m1_b, m2_w, m2_b, dn_g0, dn_be0, dn_g1, dn_be1, dn_g2, dn_be2, wn_g0, wn_be0, wn_g1, wn_be1, wn_g2, wn_be2, bnl_g, bnl_be, m1_g, m1_be, m2_g, m2_be):
    raise NotImplementedError("write your pallas kernel here")


# speedup vs baseline: 3.0164x; 3.0164x over previous
import functools
import jax
import jax.numpy as jnp
from jax import lax
from jax.experimental import pallas as pl
from jax.experimental.pallas import tpu as pltpu

B, N, S = 2, 4096, 1024
C2 = 256
CC = 128
NS = 16
BW = 0.1
IN_CH = 3 + C2
T = 256
NT = N // T
BNROWS = B * N
EPS = 1e-5
FINF = float("inf")


def _bn_cols(x, g, b):
    m = jnp.mean(x, axis=0, keepdims=True)
    v = jnp.mean((x - m) * (x - m), axis=0, keepdims=True)
    return (x - m) * lax.rsqrt(v + EPS) * g + b


def _dotT(x, w):
    return lax.dot_general(x, w, (((1,), (0,)), ((), ())),
                           preferred_element_type=jnp.float32)


def _dotX(x, w):
    return lax.dot_general(x, w, (((1,), (0,)), ((), ())),
                           precision=lax.Precision.HIGHEST,
                           preferred_element_type=jnp.float32)


def _k1_body(xyz_t_ref, xyz_f_ref, xyz2_ref, pts2_ref,
             interp_ref, gnorm_ref, idx_ref, dens_ref):
    b = pl.program_id(0)
    xt = xyz_t_ref[0]
    xf = xyz_f_ref[0]
    x2 = xyz2_ref[0]
    p2 = pts2_ref[0]

    xt_sq = jnp.sum(xt * xt, axis=1, keepdims=True)
    x2_sq = jnp.sum(x2 * x2, axis=1, keepdims=True)
    cross12 = lax.dot_general(xt, x2, (((1,), (1,)), ((), ())),
                              preferred_element_type=jnp.float32)
    d12 = xt_sq + x2_sq.T - 2.0 * cross12
    iota_s = lax.broadcasted_iota(jnp.int32, (T, S), 1)
    cur = d12
    vals = []
    idxs = []
    for _ in range(3):
        mv = jnp.min(cur, axis=1, keepdims=True)
        eq = cur == mv
        mi = jnp.min(jnp.where(eq, iota_s, S), axis=1, keepdims=True)
        vals.append(mv)
        idxs.append(mi)
        cur = jnp.where(iota_s == mi, FINF, cur)
    dists = [jnp.maximum(jnp.sqrt(jnp.maximum(v, 0.0)), 1e-10) for v in vals]
    recips = [1.0 / d for d in dists]
    norm = recips[0] + recips[1] + recips[2]
    w12 = jnp.zeros((T, S), jnp.float32)
    for mi, rc in zip(idxs, recips):
        w12 = w12 + jnp.where(iota_s == mi, rc / norm, 0.0)
    interp_ref[0] = _dotX(w12, p2)

    xf_sq = jnp.sum(xf * xf, axis=1, keepdims=True)
    cross11 = lax.dot_general(xt, xf, (((1,), (1,)), ((), ())),
                              preferred_element_type=jnp.float32)
    d11 = xt_sq + xf_sq.T - 2.0 * cross11
    gauss = jnp.exp(d11 * (-1.0 / (2.0 * BW * BW))) * (1.0 / (2.5 * BW))
    dens = jnp.mean(gauss, axis=1, keepdims=True)
    dens_ref[0] = jnp.broadcast_to(dens, (T, 8))

    iota_n = lax.broadcasted_iota(jnp.int32, (T, N), 1)
    xf_rows = [xf[:, c].reshape(1, N) for c in range(3)]
    cur = d11
    gn_cols = []
    id_cols = []
    for _ in range(NS):
        mv = jnp.min(cur, axis=1, keepdims=True)
        eq = cur == mv
        mi = jnp.min(jnp.where(eq, iota_n, N), axis=1, keepdims=True)
        oh = (iota_n == mi)
        nbr = jnp.concatenate(
            [jnp.min(jnp.where(oh, xf_rows[c], FINF), axis=1, keepdims=True)
             for c in range(3)], axis=1)
        gn_cols.append(nbr - xt)
        id_cols.append(mi + b * N)
        cur = jnp.where(oh, FINF, cur)
    gnorm_ref[0] = jnp.concatenate(gn_cols, axis=1)
    idx_ref[0] = jnp.concatenate(id_cols, axis=1)


def _run_k1(xyz1_t, xyz2_t, pts2_t):
    return pl.pallas_call(
        _k1_body,
        grid=(B, NT),
        in_specs=[
            pl.BlockSpec((1, T, 3), lambda b, i: (b, i, 0)),
            pl.BlockSpec((1, N, 3), lambda b, i: (b, 0, 0)),
            pl.BlockSpec((1, S, 3), lambda b, i: (b, 0, 0)),
            pl.BlockSpec((1, S, C2), lambda b, i: (b, 0, 0)),
        ],
        out_specs=[
            pl.BlockSpec((1, T, C2), lambda b, i: (b, i, 0)),
            pl.BlockSpec((1, T, 48), lambda b, i: (b, i, 0)),
            pl.BlockSpec((1, T, 16), lambda b, i: (b, i, 0)),
            pl.BlockSpec((1, T, 8), lambda b, i: (b, i, 0)),
        ],
        out_shape=[
            jax.ShapeDtypeStruct((B, N, C2), jnp.float32),
            jax.ShapeDtypeStruct((B, N, 48), jnp.float32),
            jax.ShapeDtypeStruct((B, N, 16), jnp.int32),
            jax.ShapeDtypeStruct((B, N, 8), jnp.float32),
        ],
    )(xyz1_t, xyz1_t, xyz2_t, pts2_t)


def _k2_body(dens_ref, w0_ref, b0_ref, w1_ref, b1_ref, w2_ref, b2_ref,
             g0_ref, be0_ref, g1_ref, be1_ref, g2_ref, be2_ref, out_ref):
    x = dens_ref[:, 0:1]
    h = x * w0_ref[0:1, :] + b0_ref[0:1, :]
    h = jax.nn.relu(_bn_cols(h, g0_ref[0:1, :], be0_ref[0:1, :]))
    h = _dotT(h, w1_ref[...]) + b1_ref[0:1, :]
    h = jax.nn.relu(_bn_cols(h, g1_ref[0:1, :], be1_ref[0:1, :]))
    h = _dotT(h, w2_ref[...]) + b2_ref[0:1, :]
    h = jax.nn.relu(_bn_cols(h, g2_ref[0:1, :], be2_ref[0:1, :]))
    out_ref[...] = jnp.broadcast_to(h, (BNROWS, 8))


def _run_k2(dens8, p):
    args = [
        dens8,
        p["dn_w0"].reshape(1, 8), p["dn_b0"].reshape(1, 8),
        p["dn_w1"].T, p["dn_b1"].reshape(1, 8),
        p["dn_w2"].T, p["dn_b2"].reshape(1, 1),
        p["dn_g0"].reshape(1, 8), p["dn_be0"].reshape(1, 8),
        p["dn_g1"].reshape(1, 8), p["dn_be1"].reshape(1, 8),
        p["dn_g2"].reshape(1, 1), p["dn_be2"].reshape(1, 1),
    ]
    return pl.pallas_call(
        _k2_body,
        out_shape=jax.ShapeDtypeStruct((BNROWS, 8), jnp.float32),
    )(*args)


def _bn_rows(x, g, b):
    m = jnp.mean(x, axis=1, keepdims=True)
    v = jnp.mean((x - m) * (x - m), axis=1, keepdims=True)
    return (x - m) * lax.rsqrt(v + EPS) * g + b


def _dotL(w, x):
    return lax.dot_general(w, x, (((1,), (0,)), ((), ())),
                           preferred_element_type=jnp.float32)


def _k3_body(gn_ref, gd_ref, w0_ref, b0_ref, w1_ref, b1_ref, w2_ref, b2_ref,
             g0_ref, be0_ref, g1_ref, be1_ref, g2_ref, be2_ref, out_ref):
    x = gn_ref[...] * gd_ref[...]
    h = _dotL(w0_ref[...], x) + b0_ref[:, 0:1]
    h = jax.nn.relu(_bn_rows(h, g0_ref[:, 0:1], be0_ref[:, 0:1]))
    h = _dotL(w1_ref[...], h) + b1_ref[:, 0:1]
    h = jax.nn.relu(_bn_rows(h, g1_ref[:, 0:1], be1_ref[:, 0:1]))
    h = _dotL(w2_ref[...], h) + b2_ref[:, 0:1]
    h = jax.nn.relu(_bn_rows(h, g2_ref[:, 0:1], be2_ref[:, 0:1]))
    out_ref[...] = h


def _run_k3(gnorm_t, gdens_row, p):
    args = [
        gnorm_t, gdens_row,
        p["wn_w0"], p["wn_b0"].reshape(8, 1),
        p["wn_w1"], p["wn_b1"].reshape(8, 1),
        p["wn_w2"], p["wn_b2"].reshape(16, 1),
        p["wn_g0"].reshape(8, 1), p["wn_be0"].reshape(8, 1),
        p["wn_g1"].reshape(8, 1), p["wn_be1"].reshape(8, 1),
        p["wn_g2"].reshape(16, 1), p["wn_be2"].reshape(16, 1),
    ]
    return pl.pallas_call(
        _k3_body,
        out_shape=jax.ShapeDtypeStruct((16, BNROWS * NS), jnp.float32),
    )(*args)


T4 = 128
NT4 = N // T4


def _k4_body(gt_ref, wt_ref, gn_ref, lw_ref, lb_ref, y_ref):
    gt = gt_ref[0]
    wt = wt_ref[0]
    gn = gn_ref[0]
    pieces = []
    for k in range(16):
        accg = jnp.zeros((T4, 3), jnp.float32)
        accf = jnp.zeros((T4, C2), jnp.float32)
        for j in range(16):
            w = wt[:, j * 16 + k:j * 16 + k + 1]
            accg = accg + gn[:, 3 * j:3 * j + 3] * w
            accf = accf + gt[:, C2 * j:C2 * (j + 1)] * w
        pieces.append(accg)
        pieces.append(accf)
    flat = jnp.concatenate(pieces, axis=1)
    y_ref[0] = _dotT(flat, lw_ref[...]) + lb_ref[0:1, :]


def _run_k4(gt, wt, gnorm, lin_w_re_t, lin_b):
    return pl.pallas_call(
        _k4_body,
        grid=(B, NT4),
        in_specs=[
            pl.BlockSpec((1, T4, NS * C2), lambda b, i: (b, i, 0)),
            pl.BlockSpec((1, T4, 256), lambda b, i: (b, i, 0)),
            pl.BlockSpec((1, T4, 48), lambda b, i: (b, i, 0)),
            pl.BlockSpec((16 * IN_CH, 256), lambda b, i: (0, 0)),
            pl.BlockSpec((1, 256), lambda b, i: (0, 0)),
        ],
        out_specs=pl.BlockSpec((1, T4, 256), lambda b, i: (b, i, 0)),
        out_shape=jax.ShapeDtypeStruct((B, N, 256), jnp.float32),
    )(gt, wt, gnorm, lin_w_re_t, lin_b)


def _k5_body(y_ref, p1_ref, bnl_g_ref, bnl_be_ref,
             w1a_ref, w1b_ref, b1_ref, g1_ref, be1_ref,
             w2_ref, b2_ref, g2_ref, be2_ref, out_ref):
    y = jax.nn.relu(_bn_cols(y_ref[...], bnl_g_ref[0:1, :], bnl_be_ref[0:1, :]))
    h = (_dotT(y, w1a_ref[...]) + _dotT(p1_ref[...], w1b_ref[...])
         + b1_ref[0:1, :])
    h = jax.nn.relu(_bn_cols(h, g1_ref[0:1, :], be1_ref[0:1, :]))
    h = _dotT(h, w2_ref[...]) + b2_ref[0:1, :]
    h = jax.nn.relu(_bn_cols(h, g2_ref[0:1, :], be2_ref[0:1, :]))
    out_ref[...] = h


def _run_k5(y1, pts1_flat, p):
    m1_wt = p["m1_w"].T
    args = [
        y1, pts1_flat,
        p["bnl_g"].reshape(1, 256), p["bnl_be"].reshape(1, 256),
        m1_wt[:256], m1_wt[256:], p["m1_b"].reshape(1, 256),
        p["m1_g"].reshape(1, 256), p["m1_be"].reshape(1, 256),
        p["m2_w"].T, p["m2_b"].reshape(1, 128),
        p["m2_g"].reshape(1, 128), p["m2_be"].reshape(1, 128),
    ]
    return pl.pallas_call(
        _k5_body,
        out_shape=jax.ShapeDtypeStruct((BNROWS, 128), jnp.float32),
    )(*args)


def kernel(xyz1, xyz2, points1, points2, dn_w0, dn_b0, dn_w1, dn_b1, dn_w2,
           dn_b2, wn_w0, wn_b0, wn_w1, wn_b1, wn_w2, wn_b2, lin_w, lin_b,
           m1_w, m1_b, m2_w, m2_b, dn_g0, dn_be0, dn_g1, dn_be1, dn_g2,
           dn_be2, wn_g0, wn_be0, wn_g1, wn_be1, wn_g2, wn_be2, bnl_g,
           bnl_be, m1_g, m1_be, m2_g, m2_be):
    p = dict(dn_w0=dn_w0, dn_b0=dn_b0, dn_w1=dn_w1, dn_b1=dn_b1,
             dn_w2=dn_w2, dn_b2=dn_b2, wn_w0=wn_w0, wn_b0=wn_b0,
             wn_w1=wn_w1, wn_b1=wn_b1, wn_w2=wn_w2, wn_b2=wn_b2,
             lin_w=lin_w, lin_b=lin_b, m1_w=m1_w, m1_b=m1_b,
             m2_w=m2_w, m2_b=m2_b, dn_g0=dn_g0, dn_be0=dn_be0,
             dn_g1=dn_g1, dn_be1=dn_be1, dn_g2=dn_g2, dn_be2=dn_be2,
             wn_g0=wn_g0, wn_be0=wn_be0, wn_g1=wn_g1, wn_be1=wn_be1,
             wn_g2=wn_g2, wn_be2=wn_be2, bnl_g=bnl_g, bnl_be=bnl_be,
             m1_g=m1_g, m1_be=m1_be, m2_g=m2_g, m2_be=m2_be)

    xyz1_t = jnp.transpose(xyz1, (0, 2, 1))
    xyz2_t = jnp.transpose(xyz2, (0, 2, 1))
    pts2_t = jnp.transpose(points2, (0, 2, 1))
    pts1_flat = jnp.transpose(points1, (0, 2, 1)).reshape(BNROWS, CC)

    interp, gnorm, idxg, dens8 = _run_k1(xyz1_t, xyz2_t, pts2_t)

    dscale8 = _run_k2(dens8.reshape(BNROWS, 8), p)
    dens_scale = dscale8[:, 0]

    idx_flat = idxg.reshape(BNROWS * NS)
    gt_flat = jnp.take(interp.reshape(BNROWS, C2), idx_flat, axis=0)
    gdens = jnp.take(dens_scale, idx_flat, axis=0)

    wt_t = _run_k3(gnorm.reshape(BNROWS * NS, 3).T,
                   gdens.reshape(1, BNROWS * NS), p)
    wt = wt_t.T

    lin_w_re_t = (lin_w.reshape(256, IN_CH, 16).transpose(2, 1, 0)
                  .reshape(16 * IN_CH, 256))
    y1 = _run_k4(gt_flat.reshape(B, N, NS * C2),
                 wt.reshape(B, N, 256), gnorm, lin_w_re_t,
                 lin_b.reshape(1, 256))

    out = _run_k5(y1.reshape(BNROWS, 256), pts1_flat, p)
    return jnp.transpose(out.reshape(B, N, 128), (0, 2, 1))
